# Initial kernel scaffold; baseline (speedup 1.0000x reference)
import jax
import jax.numpy as jnp
from jax.experimental import pallas as pl


def kernel(x, individual_labels, individual_centers):
    raise NotImplementedError("write your pallas kernel here")


# speedup vs baseline: 1.0246x; 1.0246x over previous
import functools

import jax
import jax.numpy as jnp
from jax import lax
from jax.experimental import pallas as pl
from jax.experimental.pallas import tpu as pltpu
from jax.experimental.pallas import tpu_sc as plsc

_BATCH = 4096
_FEAT = 512
_SCALE = 0.2

_NC = 2
_NS = 16
_NW = _NC * _NS
_ROWS_W = _BATCH // _NW
_CH = 32
_NCHUNK = _ROWS_W // _CH
_LANES = 16
_VPR = _FEAT // _LANES


def _body(x_hbm, labels_hbm, centers_hbm, out_hbm,
          idx0, idx1, idx2, idx3, c0, c1, x0, x1, accv,
          sc0, sc1, sx0, sx1):
    idx = (idx0, idx1, idx2, idx3)
    cbuf = (c0, c1)
    xbuf = (x0, x1)
    csem = (sc0, sc1)
    xsem = (sx0, sx1)

    wid = lax.axis_index("s") * _NC + lax.axis_index("c")
    base = wid * _ROWS_W

    for i in range(_NCHUNK):
        pltpu.sync_copy(labels_hbm.at[pl.ds(base + i * _CH, _CH)], idx[i])

    def issue(i):
        b = i % 2
        hc = pltpu.async_copy(centers_hbm.at[idx[i]], cbuf[b], csem[b])
        hx = pltpu.async_copy(x_hbm.at[pl.ds(base + i * _CH, _CH)],
                              xbuf[b], xsem[b])
        return hc, hx

    pend = issue(0)
    acc = jnp.zeros((_LANES,), jnp.float32)
    for i in range(_NCHUNK):
        b = i % 2
        hc, hx = pend
        if i + 1 < _NCHUNK:
            pend = issue(i + 1)
        hc.wait()
        hx.wait()

        def row_body(r, acc, cb=cbuf[b], xb=xbuf[b]):
            for v in range(_VPR):
                xv = xb[r, pl.ds(v * _LANES, _LANES)]
                cv = cb[r, pl.ds(v * _LANES, _LANES)]
                d = xv - cv
                acc = acc + d * d
            return acc

        acc = lax.fori_loop(0, _CH, row_body, acc)

    accv[...] = acc
    pltpu.sync_copy(accv, out_hbm.at[wid])


@jax.jit
def _partials(x, labels, centers):
    k = functools.partial(
        pl.kernel,
        mesh=plsc.VectorSubcoreMesh(core_axis_name="c", subcore_axis_name="s"),
        out_type=jax.ShapeDtypeStruct((_NW, _LANES), jnp.float32),
        scratch_types=[
            pltpu.VMEM((_CH,), jnp.int32),
            pltpu.VMEM((_CH,), jnp.int32),
            pltpu.VMEM((_CH,), jnp.int32),
            pltpu.VMEM((_CH,), jnp.int32),
            pltpu.VMEM((_CH, _FEAT), jnp.float32),
            pltpu.VMEM((_CH, _FEAT), jnp.float32),
            pltpu.VMEM((_CH, _FEAT), jnp.float32),
            pltpu.VMEM((_CH, _FEAT), jnp.float32),
            pltpu.VMEM((_LANES,), jnp.float32),
            pltpu.SemaphoreType.DMA,
            pltpu.SemaphoreType.DMA,
            pltpu.SemaphoreType.DMA,
            pltpu.SemaphoreType.DMA,
        ],
    )(_body)
    return k(x, labels, centers)


def kernel(x, individual_labels, individual_centers):
    labels = individual_labels.astype(jnp.int32)
    parts = _partials(x, labels, individual_centers)
    return (_SCALE / _BATCH) * jnp.sum(parts)


# speedup vs baseline: 1.0412x; 1.0162x over previous
import functools

import jax
import jax.numpy as jnp
from jax import lax
from jax.experimental import pallas as pl
from jax.experimental.pallas import tpu as pltpu
from jax.experimental.pallas import tpu_sc as plsc

_BATCH = 4096
_FEAT = 512
_SCALE = 0.2

_NC = 2
_NS = 16
_NW = _NC * _NS
_ROWS_W = _BATCH // _NW
_CH = 16
_NCHUNK = _ROWS_W // _CH
_NBUF = 3
_LANES = 16
_VPR = _FEAT // _LANES


def _body(x_hbm, labels_hbm, centers_hbm, out_hbm,
          idx_all, c0, c1, c2, x0, x1, x2, accv,
          sc0, sc1, sc2, sx0, sx1, sx2):
    cbuf = (c0, c1, c2)
    xbuf = (x0, x1, x2)
    csem = (sc0, sc1, sc2)
    xsem = (sx0, sx1, sx2)

    wid = lax.axis_index("s") * _NC + lax.axis_index("c")
    base = wid * _ROWS_W

    pltpu.sync_copy(labels_hbm.at[pl.ds(base, _ROWS_W)], idx_all)

    def issue(i):
        b = i % _NBUF
        hc = pltpu.async_copy(
            centers_hbm.at[idx_all.at[pl.ds(i * _CH, _CH)]], cbuf[b], csem[b])
        hx = pltpu.async_copy(x_hbm.at[pl.ds(base + i * _CH, _CH)],
                              xbuf[b], xsem[b])
        return hc, hx

    pend = [issue(i) for i in range(_NBUF - 1)]
    acc = jnp.zeros((_LANES,), jnp.float32)
    for i in range(_NCHUNK):
        b = i % _NBUF
        hc, hx = pend.pop(0)
        if i + _NBUF - 1 < _NCHUNK:
            pend.append(issue(i + _NBUF - 1))
        hc.wait()
        hx.wait()

        def row_body(r, acc, cb=cbuf[b], xb=xbuf[b]):
            for v in range(_VPR):
                xv = xb[r, pl.ds(v * _LANES, _LANES)]
                cv = cb[r, pl.ds(v * _LANES, _LANES)]
                d = xv - cv
                acc = acc + d * d
            return acc

        acc = lax.fori_loop(0, _CH, row_body, acc)

    accv[...] = acc
    pltpu.sync_copy(accv, out_hbm.at[wid])


@jax.jit
def _partials(x, labels, centers):
    k = functools.partial(
        pl.kernel,
        mesh=plsc.VectorSubcoreMesh(core_axis_name="c", subcore_axis_name="s"),
        out_type=jax.ShapeDtypeStruct((_NW, _LANES), jnp.float32),
        scratch_types=[
            pltpu.VMEM((_ROWS_W,), jnp.int32),
            pltpu.VMEM((_CH, _FEAT), jnp.float32),
            pltpu.VMEM((_CH, _FEAT), jnp.float32),
            pltpu.VMEM((_CH, _FEAT), jnp.float32),
            pltpu.VMEM((_CH, _FEAT), jnp.float32),
            pltpu.VMEM((_CH, _FEAT), jnp.float32),
            pltpu.VMEM((_CH, _FEAT), jnp.float32),
            pltpu.VMEM((_LANES,), jnp.float32),
            pltpu.SemaphoreType.DMA,
            pltpu.SemaphoreType.DMA,
            pltpu.SemaphoreType.DMA,
            pltpu.SemaphoreType.DMA,
            pltpu.SemaphoreType.DMA,
            pltpu.SemaphoreType.DMA,
        ],
    )(_body)
    return k(x, labels, centers)


def kernel(x, individual_labels, individual_centers):
    labels = individual_labels.astype(jnp.int32)
    parts = _partials(x, labels, individual_centers)
    return (_SCALE / _BATCH) * jnp.sum(parts)


# speedup vs baseline: 1.0654x; 1.0233x over previous
import functools

import jax
import jax.numpy as jnp
from jax import lax
from jax.experimental import pallas as pl
from jax.experimental.pallas import tpu as pltpu
from jax.experimental.pallas import tpu_sc as plsc

_BATCH = 4096
_FEAT = 512
_SCALE = 0.2

_NC = 2
_NS = 16
_NW = _NC * _NS
_ROWS_W = _BATCH // _NW
_CH = 16
_NCHUNK = _ROWS_W // _CH
_NBUF = 3
_LANES = 16
_VPR = _FEAT // _LANES


def _body(x_hbm, labels_hbm, centers_hbm, out_hbm,
          idx_all, c0, c1, c2, x0, x1, x2, accv,
          sc0, sc1, sc2, sx0, sx1, sx2):
    cbuf = (c0, c1, c2)
    xbuf = (x0, x1, x2)
    csem = (sc0, sc1, sc2)
    xsem = (sx0, sx1, sx2)

    wid = lax.axis_index("s") * _NC + lax.axis_index("c")
    base = wid * _ROWS_W

    pltpu.sync_copy(labels_hbm.at[pl.ds(base, _ROWS_W)], idx_all)

    def issue(i):
        b = i % _NBUF
        hc = pltpu.async_copy(
            centers_hbm.at[idx_all.at[pl.ds(i * _CH, _CH)]], cbuf[b], csem[b])
        hx = pltpu.async_copy(x_hbm.at[pl.ds(base + i * _CH, _CH)],
                              xbuf[b], xsem[b])
        return hc, hx

    pend = [issue(i) for i in range(_NBUF - 1)]
    _NACC = 8
    accs = tuple(jnp.zeros((_LANES,), jnp.float32) for _ in range(_NACC))
    for i in range(_NCHUNK):
        b = i % _NBUF
        hc, hx = pend.pop(0)
        if i + _NBUF - 1 < _NCHUNK:
            pend.append(issue(i + _NBUF - 1))
        hc.wait()
        hx.wait()

        def row_body(r, accs, cb=cbuf[b], xb=xbuf[b]):
            accs = list(accs)
            for v in range(_VPR):
                xv = xb[r, pl.ds(v * _LANES, _LANES)]
                cv = cb[r, pl.ds(v * _LANES, _LANES)]
                d = xv - cv
                accs[v % _NACC] = accs[v % _NACC] + d * d
            return tuple(accs)

        accs = lax.fori_loop(0, _CH, row_body, accs)

    acc = ((accs[0] + accs[1]) + (accs[2] + accs[3])) + \
          ((accs[4] + accs[5]) + (accs[6] + accs[7]))
    accv[...] = acc
    pltpu.sync_copy(accv, out_hbm.at[wid])


@jax.jit
def _partials(x, labels, centers):
    k = functools.partial(
        pl.kernel,
        mesh=plsc.VectorSubcoreMesh(core_axis_name="c", subcore_axis_name="s"),
        out_type=jax.ShapeDtypeStruct((_NW, _LANES), jnp.float32),
        scratch_types=[
            pltpu.VMEM((_ROWS_W,), jnp.int32),
            pltpu.VMEM((_CH, _FEAT), jnp.float32),
            pltpu.VMEM((_CH, _FEAT), jnp.float32),
            pltpu.VMEM((_CH, _FEAT), jnp.float32),
            pltpu.VMEM((_CH, _FEAT), jnp.float32),
            pltpu.VMEM((_CH, _FEAT), jnp.float32),
            pltpu.VMEM((_CH, _FEAT), jnp.float32),
            pltpu.VMEM((_LANES,), jnp.float32),
            pltpu.SemaphoreType.DMA,
            pltpu.SemaphoreType.DMA,
            pltpu.SemaphoreType.DMA,
            pltpu.SemaphoreType.DMA,
            pltpu.SemaphoreType.DMA,
            pltpu.SemaphoreType.DMA,
        ],
    )(_body)
    return k(x, labels, centers)


def kernel(x, individual_labels, individual_centers):
    labels = individual_labels.astype(jnp.int32)
    parts = _partials(x, labels, individual_centers)
    return (_SCALE / _BATCH) * jnp.sum(parts)


# speedup vs baseline: 1.0668x; 1.0014x over previous
import functools

import jax
import jax.numpy as jnp
from jax import lax
from jax.experimental import pallas as pl
from jax.experimental.pallas import tpu as pltpu
from jax.experimental.pallas import tpu_sc as plsc

_BATCH = 4096
_FEAT = 512
_SCALE = 0.2

_NC = 2
_NS = 16
_NW = _NC * _NS
_ROWS_W = _BATCH // _NW
_CH = 16
_NCHUNK = _ROWS_W // _CH
_NBUF = 3
_LANES = 16
_VPR = _FEAT // _LANES


def _body(x_hbm, labels_hbm, centers_hbm, out_hbm,
          idx_all, c0, c1, c2, x0, x1, x2, accv,
          sc0, sc1, sc2, sx0, sx1, sx2):
    cbuf = (c0, c1, c2)
    xbuf = (x0, x1, x2)
    csem = (sc0, sc1, sc2)
    xsem = (sx0, sx1, sx2)

    wid = lax.axis_index("s") * _NC + lax.axis_index("c")
    base = wid * _ROWS_W

    pltpu.sync_copy(labels_hbm.at[pl.ds(base, _ROWS_W)], idx_all)

    def issue(i):
        b = i % _NBUF
        hc = pltpu.async_copy(
            centers_hbm.at[idx_all.at[pl.ds(i * _CH, _CH)]], cbuf[b], csem[b])
        hx = pltpu.async_copy(x_hbm.at[pl.ds(base + i * _CH, _CH)],
                              xbuf[b], xsem[b])
        return hc, hx

    pend = [issue(i) for i in range(_NBUF - 1)]
    _NACC = 8
    accs = tuple(jnp.zeros((_LANES,), jnp.float32) for _ in range(_NACC))
    for i in range(_NCHUNK):
        b = i % _NBUF
        hc, hx = pend.pop(0)
        if i + _NBUF - 1 < _NCHUNK:
            pend.append(issue(i + _NBUF - 1))
        hc.wait()
        hx.wait()

        def row_body(r, accs, cb=cbuf[b], xb=xbuf[b]):
            accs = list(accs)
            for v in range(_VPR):
                xv = xb[r, pl.ds(v * _LANES, _LANES)]
                cv = cb[r, pl.ds(v * _LANES, _LANES)]
                d = xv - cv
                accs[v % _NACC] = accs[v % _NACC] + d * d
            return tuple(accs)

        accs = plsc.parallel_loop(0, _CH, unroll=2, carry=accs)(row_body)

    acc = ((accs[0] + accs[1]) + (accs[2] + accs[3])) + \
          ((accs[4] + accs[5]) + (accs[6] + accs[7]))
    accv[...] = acc
    pltpu.sync_copy(accv, out_hbm.at[wid])


@jax.jit
def _partials(x, labels, centers):
    k = functools.partial(
        pl.kernel,
        mesh=plsc.VectorSubcoreMesh(core_axis_name="c", subcore_axis_name="s"),
        out_type=jax.ShapeDtypeStruct((_NW, _LANES), jnp.float32),
        scratch_types=[
            pltpu.VMEM((_ROWS_W,), jnp.int32),
            pltpu.VMEM((_CH, _FEAT), jnp.float32),
            pltpu.VMEM((_CH, _FEAT), jnp.float32),
            pltpu.VMEM((_CH, _FEAT), jnp.float32),
            pltpu.VMEM((_CH, _FEAT), jnp.float32),
            pltpu.VMEM((_CH, _FEAT), jnp.float32),
            pltpu.VMEM((_CH, _FEAT), jnp.float32),
            pltpu.VMEM((_LANES,), jnp.float32),
            pltpu.SemaphoreType.DMA,
            pltpu.SemaphoreType.DMA,
            pltpu.SemaphoreType.DMA,
            pltpu.SemaphoreType.DMA,
            pltpu.SemaphoreType.DMA,
            pltpu.SemaphoreType.DMA,
        ],
    )(_body)
    return k(x, labels, centers)


def kernel(x, individual_labels, individual_centers):
    labels = individual_labels.astype(jnp.int32)
    parts = _partials(x, labels, individual_centers)
    return (_SCALE / _BATCH) * jnp.sum(parts)


# speedup vs baseline: 1.1206x; 1.0504x over previous
import functools

import jax
import jax.numpy as jnp
from jax import lax
from jax.experimental import pallas as pl
from jax.experimental.pallas import tpu as pltpu
from jax.experimental.pallas import tpu_sc as plsc

_BATCH = 4096
_FEAT = 512
_SCALE = 0.2

_NC = 2
_NS = 16
_NW = _NC * _NS
_ROWS_W = _BATCH // _NW
_CH = 16
_NCHUNK = _ROWS_W // _CH
_NBUF = 4
_NGRP = _NCHUNK // _NBUF
_LANES = 16
_VPR = _FEAT // _LANES


def _body(x_hbm, labels_hbm, centers_hbm, out_hbm,
          idx_all, c0, c1, c2, c3, x0, x1, x2, x3, accv,
          sc0, sc1, sc2, sc3, sx0, sx1, sx2, sx3):
    cbuf = (c0, c1, c2, c3)
    xbuf = (x0, x1, x2, x3)
    csem = (sc0, sc1, sc2, sc3)
    xsem = (sx0, sx1, sx2, sx3)

    wid = lax.axis_index("s") * _NC + lax.axis_index("c")
    base = wid * _ROWS_W

    pltpu.sync_copy(labels_hbm.at[pl.ds(base, _ROWS_W)], idx_all)

    def issue(i, b):
        pltpu.async_copy(
            centers_hbm.at[idx_all.at[pl.ds(i * _CH, _CH)]], cbuf[b], csem[b])
        pltpu.async_copy(x_hbm.at[pl.ds(base + i * _CH, _CH)],
                         xbuf[b], xsem[b])

    def wait(b):
        pltpu.make_async_copy(x_hbm.at[pl.ds(base, _CH)], cbuf[b], csem[b]).wait()
        pltpu.make_async_copy(x_hbm.at[pl.ds(base, _CH)], xbuf[b], xsem[b]).wait()

    for i in range(_NBUF - 1):
        issue(i, i)

    _NACC = 8
    accs = tuple(jnp.zeros((_LANES,), jnp.float32) for _ in range(_NACC))

    def grp_body(g, accs):
        for p in range(_NBUF):
            j = g * _NBUF + p + (_NBUF - 1)

            @pl.when(j < _NCHUNK)
            def _(j=j, p=p):
                issue(j, (p + _NBUF - 1) % _NBUF)

            wait(p)

            def row_body(r, accs, cb=cbuf[p], xb=xbuf[p]):
                accs = list(accs)
                for v in range(_VPR):
                    xv = xb[r, pl.ds(v * _LANES, _LANES)]
                    cv = cb[r, pl.ds(v * _LANES, _LANES)]
                    d = xv - cv
                    accs[v % _NACC] = accs[v % _NACC] + d * d
                return tuple(accs)

            accs = plsc.parallel_loop(0, _CH, unroll=2, carry=accs)(row_body)
        return accs

    accs = lax.fori_loop(0, _NGRP, grp_body, accs)

    acc = ((accs[0] + accs[1]) + (accs[2] + accs[3])) + \
          ((accs[4] + accs[5]) + (accs[6] + accs[7]))
    accv[...] = acc
    pltpu.sync_copy(accv, out_hbm.at[wid])


@jax.jit
def _partials(x, labels, centers):
    k = functools.partial(
        pl.kernel,
        mesh=plsc.VectorSubcoreMesh(core_axis_name="c", subcore_axis_name="s"),
        out_type=jax.ShapeDtypeStruct((_NW, _LANES), jnp.float32),
        scratch_types=[
            pltpu.VMEM((_ROWS_W,), jnp.int32),
            pltpu.VMEM((_CH, _FEAT), jnp.float32),
            pltpu.VMEM((_CH, _FEAT), jnp.float32),
            pltpu.VMEM((_CH, _FEAT), jnp.float32),
            pltpu.VMEM((_CH, _FEAT), jnp.float32),
            pltpu.VMEM((_CH, _FEAT), jnp.float32),
            pltpu.VMEM((_CH, _FEAT), jnp.float32),
            pltpu.VMEM((_CH, _FEAT), jnp.float32),
            pltpu.VMEM((_CH, _FEAT), jnp.float32),
            pltpu.VMEM((_LANES,), jnp.float32),
            pltpu.SemaphoreType.DMA,
            pltpu.SemaphoreType.DMA,
            pltpu.SemaphoreType.DMA,
            pltpu.SemaphoreType.DMA,
            pltpu.SemaphoreType.DMA,
            pltpu.SemaphoreType.DMA,
            pltpu.SemaphoreType.DMA,
            pltpu.SemaphoreType.DMA,
        ],
    )(_body)
    return k(x, labels, centers)


def kernel(x, individual_labels, individual_centers):
    labels = individual_labels.astype(jnp.int32)
    parts = _partials(x, labels, individual_centers)
    return (_SCALE / _BATCH) * jnp.sum(parts)


# speedup vs baseline: 1.1212x; 1.0005x over previous
import functools

import jax
import jax.numpy as jnp
from jax import lax
from jax.experimental import pallas as pl
from jax.experimental.pallas import tpu as pltpu
from jax.experimental.pallas import tpu_sc as plsc

_BATCH = 4096
_FEAT = 512
_SCALE = 0.2

_NC = 2
_NS = 16
_NW = _NC * _NS
_ROWS_W = _BATCH // _NW
_CH = 16
_NCHUNK = _ROWS_W // _CH
_NBUF = 4
_NGRP = _NCHUNK // _NBUF
_LANES = 16
_VPR = _FEAT // _LANES


def _body(x_hbm, labels_hbm, centers_hbm, out_hbm,
          idx_all, c0, c1, c2, c3, x0, x1, x2, x3, accv,
          sc0, sc1, sc2, sc3, sx0, sx1, sx2, sx3):
    cbuf = (c0, c1, c2, c3)
    xbuf = (x0, x1, x2, x3)
    csem = (sc0, sc1, sc2, sc3)
    xsem = (sx0, sx1, sx2, sx3)

    wid = lax.axis_index("s") * _NC + lax.axis_index("c")
    base = wid * _ROWS_W

    pltpu.sync_copy(labels_hbm.at[pl.ds(base, _ROWS_W)], idx_all)

    def issue(i, b):
        pltpu.async_copy(
            centers_hbm.at[idx_all.at[pl.ds(i * _CH, _CH)]], cbuf[b], csem[b])
        pltpu.async_copy(x_hbm.at[pl.ds(base + i * _CH, _CH)],
                         xbuf[b], xsem[b])

    def wait(b):
        pltpu.make_async_copy(x_hbm.at[pl.ds(base, _CH)], cbuf[b], csem[b]).wait()
        pltpu.make_async_copy(x_hbm.at[pl.ds(base, _CH)], xbuf[b], xsem[b]).wait()

    for i in range(_NBUF - 1):
        issue(i, i)

    _NACC = 8
    accs = tuple(jnp.zeros((_LANES,), jnp.float32) for _ in range(_NACC))

    def grp_body(g, accs):
        for p in range(_NBUF):
            j = g * _NBUF + p + (_NBUF - 1)

            @pl.when(j < _NCHUNK)
            def _(j=j, p=p):
                issue(j, (p + _NBUF - 1) % _NBUF)

            wait(p)

            def row_body(r, accs, cb=cbuf[p], xb=xbuf[p]):
                accs = list(accs)
                for v in range(_VPR):
                    xv = xb[r, pl.ds(v * _LANES, _LANES)]
                    cv = cb[r, pl.ds(v * _LANES, _LANES)]
                    d = xv - cv
                    accs[v % _NACC] = accs[v % _NACC] + d * d
                return tuple(accs)

            accs = plsc.parallel_loop(0, _CH, unroll=4, carry=accs)(row_body)
        return accs

    accs = lax.fori_loop(0, _NGRP, grp_body, accs)

    acc = ((accs[0] + accs[1]) + (accs[2] + accs[3])) + \
          ((accs[4] + accs[5]) + (accs[6] + accs[7]))
    accv[...] = acc
    pltpu.sync_copy(accv, out_hbm.at[wid])


@jax.jit
def _partials(x, labels, centers):
    k = functools.partial(
        pl.kernel,
        mesh=plsc.VectorSubcoreMesh(core_axis_name="c", subcore_axis_name="s"),
        out_type=jax.ShapeDtypeStruct((_NW, _LANES), jnp.float32),
        scratch_types=[
            pltpu.VMEM((_ROWS_W,), jnp.int32),
            pltpu.VMEM((_CH, _FEAT), jnp.float32),
            pltpu.VMEM((_CH, _FEAT), jnp.float32),
            pltpu.VMEM((_CH, _FEAT), jnp.float32),
            pltpu.VMEM((_CH, _FEAT), jnp.float32),
            pltpu.VMEM((_CH, _FEAT), jnp.float32),
            pltpu.VMEM((_CH, _FEAT), jnp.float32),
            pltpu.VMEM((_CH, _FEAT), jnp.float32),
            pltpu.VMEM((_CH, _FEAT), jnp.float32),
            pltpu.VMEM((_LANES,), jnp.float32),
            pltpu.SemaphoreType.DMA,
            pltpu.SemaphoreType.DMA,
            pltpu.SemaphoreType.DMA,
            pltpu.SemaphoreType.DMA,
            pltpu.SemaphoreType.DMA,
            pltpu.SemaphoreType.DMA,
            pltpu.SemaphoreType.DMA,
            pltpu.SemaphoreType.DMA,
        ],
    )(_body)
    return k(x, labels, centers)


def kernel(x, individual_labels, individual_centers):
    labels = individual_labels.astype(jnp.int32)
    parts = _partials(x, labels, individual_centers)
    return (_SCALE / _BATCH) * jnp.sum(parts)


# speedup vs baseline: 1.1321x; 1.0098x over previous
import functools

import jax
import jax.numpy as jnp
from jax import lax
from jax.experimental import pallas as pl
from jax.experimental.pallas import tpu as pltpu
from jax.experimental.pallas import tpu_sc as plsc

_BATCH = 4096
_FEAT = 512
_SCALE = 0.2

_NC = 2
_NS = 16
_NW = _NC * _NS
_ROWS_W = _BATCH // _NW
_CH = 8
_NCHUNK = _ROWS_W // _CH
_NBUF = 4
_NGRP = _NCHUNK // _NBUF
_LANES = 16
_VPR = _FEAT // _LANES


def _body(x_hbm, labels_hbm, centers_hbm, out_hbm,
          idx_all, c0, c1, c2, c3, x0, x1, x2, x3, accv,
          sc0, sc1, sc2, sc3, sx0, sx1, sx2, sx3):
    cbuf = (c0, c1, c2, c3)
    xbuf = (x0, x1, x2, x3)
    csem = (sc0, sc1, sc2, sc3)
    xsem = (sx0, sx1, sx2, sx3)

    wid = lax.axis_index("s") * _NC + lax.axis_index("c")
    base = wid * _ROWS_W

    pltpu.sync_copy(labels_hbm.at[pl.ds(base, _ROWS_W)], idx_all)

    def issue(i, b):
        pltpu.async_copy(
            centers_hbm.at[idx_all.at[pl.ds(i * _CH, _CH)]], cbuf[b], csem[b])
        pltpu.async_copy(x_hbm.at[pl.ds(base + i * _CH, _CH)],
                         xbuf[b], xsem[b])

    def wait(b):
        pltpu.make_async_copy(x_hbm.at[pl.ds(base, _CH)], cbuf[b], csem[b]).wait()
        pltpu.make_async_copy(x_hbm.at[pl.ds(base, _CH)], xbuf[b], xsem[b]).wait()

    for i in range(_NBUF - 1):
        issue(i, i)

    _NACC = 8
    accs = tuple(jnp.zeros((_LANES,), jnp.float32) for _ in range(_NACC))

    def grp_body(g, accs):
        for p in range(_NBUF):
            j = g * _NBUF + p + (_NBUF - 1)

            @pl.when(j < _NCHUNK)
            def _(j=j, p=p):
                issue(j, (p + _NBUF - 1) % _NBUF)

            wait(p)

            def row_body(r, accs, cb=cbuf[p], xb=xbuf[p]):
                accs = list(accs)
                for v in range(_VPR):
                    xv = xb[r, pl.ds(v * _LANES, _LANES)]
                    cv = cb[r, pl.ds(v * _LANES, _LANES)]
                    d = xv - cv
                    accs[v % _NACC] = accs[v % _NACC] + d * d
                return tuple(accs)

            accs = plsc.parallel_loop(0, _CH, unroll=2, carry=accs)(row_body)
        return accs

    accs = lax.fori_loop(0, _NGRP, grp_body, accs)

    acc = ((accs[0] + accs[1]) + (accs[2] + accs[3])) + \
          ((accs[4] + accs[5]) + (accs[6] + accs[7]))
    accv[...] = acc
    pltpu.sync_copy(accv, out_hbm.at[wid])


@jax.jit
def _partials(x, labels, centers):
    k = functools.partial(
        pl.kernel,
        mesh=plsc.VectorSubcoreMesh(core_axis_name="c", subcore_axis_name="s"),
        out_type=jax.ShapeDtypeStruct((_NW, _LANES), jnp.float32),
        scratch_types=[
            pltpu.VMEM((_ROWS_W,), jnp.int32),
            pltpu.VMEM((_CH, _FEAT), jnp.float32),
            pltpu.VMEM((_CH, _FEAT), jnp.float32),
            pltpu.VMEM((_CH, _FEAT), jnp.float32),
            pltpu.VMEM((_CH, _FEAT), jnp.float32),
            pltpu.VMEM((_CH, _FEAT), jnp.float32),
            pltpu.VMEM((_CH, _FEAT), jnp.float32),
            pltpu.VMEM((_CH, _FEAT), jnp.float32),
            pltpu.VMEM((_CH, _FEAT), jnp.float32),
            pltpu.VMEM((_LANES,), jnp.float32),
            pltpu.SemaphoreType.DMA,
            pltpu.SemaphoreType.DMA,
            pltpu.SemaphoreType.DMA,
            pltpu.SemaphoreType.DMA,
            pltpu.SemaphoreType.DMA,
            pltpu.SemaphoreType.DMA,
            pltpu.SemaphoreType.DMA,
            pltpu.SemaphoreType.DMA,
        ],
    )(_body)
    return k(x, labels, centers)


def kernel(x, individual_labels, individual_centers):
    labels = individual_labels.astype(jnp.int32)
    parts = _partials(x, labels, individual_centers)
    return (_SCALE / _BATCH) * jnp.sum(parts)
